# baseline (speedup 1.0000x reference)
import functools

import jax
import jax.numpy as jnp
from jax import lax
from jax.experimental import pallas as pl
from jax.experimental.pallas import tpu as pltpu
from jax.experimental.pallas import tpu_sc as plsc

NE = 8
NB = 2
D = 2048
B = 16
T = 32
S = NB * B
TJ = 512
NJ = D // TJ
L = 16


def _logits_kernel(x_ref, gw_ref, lg_ref):
    x = x_ref[...]
    xavg = jnp.sum(x, axis=1) * (1.0 / T)
    lg_ref[...] = jax.lax.dot_general(
        gw_ref[...], xavg,
        dimension_numbers=(((1,), (1,)), ((), ())),
        preferred_element_type=jnp.float32)


def _route_sc_kernel(lg_hbm, sc_hbm, rt_hbm, lg_v, sc_v, rt_v):
    wid = lax.axis_index("s") * 2 + lax.axis_index("c")

    @pl.when(wid == 0)
    def _():
        pltpu.sync_copy(lg_hbm, lg_v)
        rows = [lg_v[pl.ds(e * L, L)] for e in range(NE)]
        m = rows[0]
        for e in range(1, NE):
            m = jnp.maximum(m, rows[e])
        exs = [jnp.exp(r - m) for r in rows]
        tot = exs[0]
        for e in range(1, NE):
            tot = tot + exs[e]
        ps = [ex / tot for ex in exs]
        v0, i0 = ps[0], jnp.zeros((L,), jnp.int32)
        for e in range(1, NE):
            gt = ps[e] > v0
            v0 = jnp.where(gt, ps[e], v0)
            i0 = jnp.where(gt, jnp.full((L,), e, jnp.int32), i0)
        p2 = [jnp.where(i0 == e, -1.0, ps[e]) for e in range(NE)]
        v1, i1 = p2[0], jnp.zeros((L,), jnp.int32)
        for e in range(1, NE):
            gt = p2[e] > v1
            v1 = jnp.where(gt, p2[e], v1)
            i1 = jnp.where(gt, jnp.full((L,), e, jnp.int32), i1)
        sc_v[pl.ds(0, L)] = v0
        sc_v[pl.ds(L, L)] = v1
        rt_v[pl.ds(0, L)] = i0
        rt_v[pl.ds(L, L)] = i1
        pltpu.sync_copy(sc_v, sc_hbm)
        pltpu.sync_copy(rt_v, rt_hbm)


def _route(logits_t):
    mesh = plsc.VectorSubcoreMesh(core_axis_name="c", subcore_axis_name="s")
    f = functools.partial(
        pl.kernel, mesh=mesh,
        out_type=[
            jax.ShapeDtypeStruct((S,), jnp.float32),
            jax.ShapeDtypeStruct((S,), jnp.int32),
        ],
        scratch_types=[
            pltpu.VMEM((NE * L,), jnp.float32),
            pltpu.VMEM((S,), jnp.float32),
            pltpu.VMEM((S,), jnp.int32),
        ],
    )(_route_sc_kernel)
    return f(logits_t.reshape(NE * B))


def _ffn_kernel(x_ref, topi_ref, w_ref, b_ref, out_ref):
    e = pl.program_id(1)
    t = jax.lax.dot_general(
        x_ref[...], w_ref[0].astype(jnp.bfloat16),
        dimension_numbers=(((1,), (1,)), ((), ())),
        preferred_element_type=jnp.float32)
    t3 = t.reshape(B, T, TJ) + b_ref[0][None]
    m0 = (topi_ref[...][:, 0:1] == e)[:, :, None]
    m1 = (topi_ref[...][:, 1:2] == e)[:, :, None]
    c0 = m0.astype(jnp.float32) * t3
    c1 = m1.astype(jnp.float32) * t3

    @pl.when(e == 0)
    def _():
        out_ref[:, :T, :] = c0
        out_ref[:, T:, :] = c1

    @pl.when(e != 0)
    def _():
        out_ref[:, :T, :] += c0
        out_ref[:, T:, :] += c1


def kernel(x, attention_mask, gate_w, expert_w, expert_b):
    logits_t = pl.pallas_call(
        _logits_kernel,
        out_shape=jax.ShapeDtypeStruct((NE, B), jnp.float32),
    )(x, gate_w)

    scores2, route2 = _route(logits_t)
    topi = route2.reshape(NB, B).T
    beam_scores = scores2.reshape(NB, B).T.reshape(S)
    expert_route = topi.reshape(S)[:, None]

    xf = x.reshape(B * T, D).astype(jnp.bfloat16)
    out = pl.pallas_call(
        _ffn_kernel,
        grid=(NJ, NE),
        in_specs=[
            pl.BlockSpec((B * T, D), lambda j, e: (0, 0)),
            pl.BlockSpec((B, NB), lambda j, e: (0, 0)),
            pl.BlockSpec((1, TJ, D), lambda j, e: (e, j, 0)),
            pl.BlockSpec((1, 1, TJ), lambda j, e: (e, 0, j)),
        ],
        out_specs=pl.BlockSpec((B, NB * T, TJ), lambda j, e: (0, 0, j)),
        out_shape=jax.ShapeDtypeStruct((B, NB * T, D), jnp.float32),
        compiler_params=pltpu.CompilerParams(
            dimension_semantics=("arbitrary", "arbitrary")),
    )(xf, topi, expert_w, expert_b.reshape(NE, 1, D))

    candidate_output = out.reshape(S, T, D)
    return candidate_output, beam_scores, expert_route


# speedup vs baseline: 1.1109x; 1.1109x over previous
import functools

import jax
import jax.numpy as jnp
from jax import lax
from jax.experimental import pallas as pl
from jax.experimental.pallas import tpu as pltpu
from jax.experimental.pallas import tpu_sc as plsc

NE = 8
NB = 2
D = 2048
B = 16
T = 32
S = NB * B
TJ = 2048
NJ = D // TJ
L = 16


def _logits_kernel(x_ref, gw_ref, lg_ref):
    x = x_ref[...]
    xavg = jnp.sum(x, axis=1) * (1.0 / T)
    lg_ref[...] = jax.lax.dot_general(
        gw_ref[...], xavg,
        dimension_numbers=(((1,), (1,)), ((), ())),
        preferred_element_type=jnp.float32)


def _route_sc_kernel(lg_hbm, sc_hbm, rt_hbm, lg_v, sc_v, rt_v):
    wid = lax.axis_index("s") * 2 + lax.axis_index("c")

    @pl.when(wid == 0)
    def _():
        pltpu.sync_copy(lg_hbm, lg_v)
        rows = [lg_v[pl.ds(e * L, L)] for e in range(NE)]
        m = rows[0]
        for e in range(1, NE):
            m = jnp.maximum(m, rows[e])
        exs = [jnp.exp(r - m) for r in rows]
        tot = exs[0]
        for e in range(1, NE):
            tot = tot + exs[e]
        ps = [ex / tot for ex in exs]
        v0, i0 = ps[0], jnp.zeros((L,), jnp.int32)
        for e in range(1, NE):
            gt = ps[e] > v0
            v0 = jnp.where(gt, ps[e], v0)
            i0 = jnp.where(gt, jnp.full((L,), e, jnp.int32), i0)
        p2 = [jnp.where(i0 == e, -1.0, ps[e]) for e in range(NE)]
        v1, i1 = p2[0], jnp.zeros((L,), jnp.int32)
        for e in range(1, NE):
            gt = p2[e] > v1
            v1 = jnp.where(gt, p2[e], v1)
            i1 = jnp.where(gt, jnp.full((L,), e, jnp.int32), i1)
        sc_v[pl.ds(0, L)] = v0
        sc_v[pl.ds(L, L)] = v1
        rt_v[pl.ds(0, L)] = i0
        rt_v[pl.ds(L, L)] = i1
        pltpu.sync_copy(sc_v, sc_hbm)
        pltpu.sync_copy(rt_v, rt_hbm)


def _route(logits_t):
    mesh = plsc.VectorSubcoreMesh(core_axis_name="c", subcore_axis_name="s")
    f = functools.partial(
        pl.kernel, mesh=mesh,
        out_type=[
            jax.ShapeDtypeStruct((S,), jnp.float32),
            jax.ShapeDtypeStruct((S,), jnp.int32),
        ],
        scratch_types=[
            pltpu.VMEM((NE * L,), jnp.float32),
            pltpu.VMEM((S,), jnp.float32),
            pltpu.VMEM((S,), jnp.int32),
        ],
    )(_route_sc_kernel)
    return f(logits_t.reshape(NE * B))


def _ffn_kernel(x_ref, topi_ref, w_ref, b_ref, out_ref):
    e = pl.program_id(1)
    t = jax.lax.dot_general(
        x_ref[...], w_ref[0].astype(jnp.bfloat16),
        dimension_numbers=(((1,), (1,)), ((), ())),
        preferred_element_type=jnp.float32)
    t3 = t.reshape(B, T, TJ) + b_ref[0][None]
    m0 = (topi_ref[...][:, 0:1] == e)[:, :, None]
    m1 = (topi_ref[...][:, 1:2] == e)[:, :, None]
    c0 = m0.astype(jnp.float32) * t3
    c1 = m1.astype(jnp.float32) * t3

    @pl.when(e == 0)
    def _():
        out_ref[:, :T, :] = c0
        out_ref[:, T:, :] = c1

    @pl.when(e != 0)
    def _():
        out_ref[:, :T, :] += c0
        out_ref[:, T:, :] += c1


def kernel(x, attention_mask, gate_w, expert_w, expert_b):
    logits_t = pl.pallas_call(
        _logits_kernel,
        out_shape=jax.ShapeDtypeStruct((NE, B), jnp.float32),
    )(x, gate_w)

    scores2, route2 = _route(logits_t)
    topi = route2.reshape(NB, B).T
    beam_scores = scores2.reshape(NB, B).T.reshape(S)
    expert_route = topi.reshape(S)[:, None]

    xf = x.reshape(B * T, D).astype(jnp.bfloat16)
    out = pl.pallas_call(
        _ffn_kernel,
        grid=(NJ, NE),
        in_specs=[
            pl.BlockSpec((B * T, D), lambda j, e: (0, 0)),
            pl.BlockSpec((B, NB), lambda j, e: (0, 0)),
            pl.BlockSpec((1, TJ, D), lambda j, e: (e, j, 0)),
            pl.BlockSpec((1, 1, TJ), lambda j, e: (e, 0, j)),
        ],
        out_specs=pl.BlockSpec((B, NB * T, TJ), lambda j, e: (0, 0, j)),
        out_shape=jax.ShapeDtypeStruct((B, NB * T, D), jnp.float32),
        compiler_params=pltpu.CompilerParams(
            dimension_semantics=("arbitrary", "arbitrary")),
    )(xf, topi, expert_w, expert_b.reshape(NE, 1, D))

    candidate_output = out.reshape(S, T, D)
    return candidate_output, beam_scores, expert_route


# speedup vs baseline: 1.1302x; 1.0174x over previous
import functools

import jax
import jax.numpy as jnp
from jax import lax
from jax.experimental import pallas as pl
from jax.experimental.pallas import tpu as pltpu
from jax.experimental.pallas import tpu_sc as plsc

NE = 8
NB = 2
D = 2048
B = 16
T = 32
S = NB * B
TJ = 2048
NJ = D // TJ
L = 16


def _logits_kernel(x_ref, gw_ref, lg_ref, xf_ref):
    x = x_ref[...]
    xavg = jnp.sum(x, axis=1) * (1.0 / T)
    lg_ref[...] = jax.lax.dot_general(
        gw_ref[...], xavg,
        dimension_numbers=(((1,), (1,)), ((), ())),
        preferred_element_type=jnp.float32)
    xf_ref[...] = x.reshape(B * T, D).astype(jnp.bfloat16)


def _route_sc_kernel(lg_hbm, sc_hbm, rt_hbm, lg_v, sc_v, rt_v):
    wid = lax.axis_index("s") * 2 + lax.axis_index("c")

    @pl.when(wid == 0)
    def _():
        pltpu.sync_copy(lg_hbm, lg_v)
        rows = [lg_v[pl.ds(e * L, L)] for e in range(NE)]
        m = rows[0]
        for e in range(1, NE):
            m = jnp.maximum(m, rows[e])
        exs = [jnp.exp(r - m) for r in rows]
        tot = exs[0]
        for e in range(1, NE):
            tot = tot + exs[e]
        ps = [ex / tot for ex in exs]
        v0, i0 = ps[0], jnp.zeros((L,), jnp.int32)
        for e in range(1, NE):
            gt = ps[e] > v0
            v0 = jnp.where(gt, ps[e], v0)
            i0 = jnp.where(gt, jnp.full((L,), e, jnp.int32), i0)
        p2 = [jnp.where(i0 == e, -1.0, ps[e]) for e in range(NE)]
        v1, i1 = p2[0], jnp.zeros((L,), jnp.int32)
        for e in range(1, NE):
            gt = p2[e] > v1
            v1 = jnp.where(gt, p2[e], v1)
            i1 = jnp.where(gt, jnp.full((L,), e, jnp.int32), i1)
        sc_v[pl.ds(0, L)] = v0
        sc_v[pl.ds(L, L)] = v1
        rt_v[pl.ds(0, L)] = i0
        rt_v[pl.ds(L, L)] = i1
        pltpu.sync_copy(sc_v, sc_hbm)
        pltpu.sync_copy(rt_v, rt_hbm)


def _route(logits_t):
    mesh = plsc.VectorSubcoreMesh(core_axis_name="c", subcore_axis_name="s")
    f = functools.partial(
        pl.kernel, mesh=mesh,
        out_type=[
            jax.ShapeDtypeStruct((S,), jnp.float32),
            jax.ShapeDtypeStruct((S,), jnp.int32),
        ],
        scratch_types=[
            pltpu.VMEM((NE * L,), jnp.float32),
            pltpu.VMEM((S,), jnp.float32),
            pltpu.VMEM((S,), jnp.int32),
        ],
    )(_route_sc_kernel)
    return f(logits_t.reshape(NE * B))


def _ffn_kernel(x_ref, topi_ref, w_ref, b_ref, out_ref):
    e = pl.program_id(1)
    t = jax.lax.dot_general(
        x_ref[...], w_ref[0].astype(jnp.bfloat16),
        dimension_numbers=(((1,), (1,)), ((), ())),
        preferred_element_type=jnp.float32)
    t3 = t.reshape(B, T, TJ) + b_ref[0][None]
    m0 = (topi_ref[...][:, 0:1] == e)[:, :, None]
    m1 = (topi_ref[...][:, 1:2] == e)[:, :, None]
    c0 = m0.astype(jnp.float32) * t3
    c1 = m1.astype(jnp.float32) * t3

    @pl.when(e == 0)
    def _():
        out_ref[:, :T, :] = c0
        out_ref[:, T:, :] = c1

    @pl.when(e != 0)
    def _():
        out_ref[:, :T, :] += c0
        out_ref[:, T:, :] += c1


def kernel(x, attention_mask, gate_w, expert_w, expert_b):
    logits_t, xf = pl.pallas_call(
        _logits_kernel,
        out_shape=[
            jax.ShapeDtypeStruct((NE, B), jnp.float32),
            jax.ShapeDtypeStruct((B * T, D), jnp.bfloat16),
        ],
    )(x, gate_w)

    scores2, route2 = _route(logits_t)
    topi = route2.reshape(NB, B).T
    beam_scores = scores2.reshape(NB, B).T.reshape(S)
    expert_route = topi.reshape(S)[:, None]

    out = pl.pallas_call(
        _ffn_kernel,
        grid=(NJ, NE),
        in_specs=[
            pl.BlockSpec((B * T, D), lambda j, e: (0, 0)),
            pl.BlockSpec((B, NB), lambda j, e: (0, 0)),
            pl.BlockSpec((1, TJ, D), lambda j, e: (e, j, 0)),
            pl.BlockSpec((1, 1, TJ), lambda j, e: (e, 0, j)),
        ],
        out_specs=pl.BlockSpec((B, NB * T, TJ), lambda j, e: (0, 0, j)),
        out_shape=jax.ShapeDtypeStruct((B, NB * T, D), jnp.float32),
        compiler_params=pltpu.CompilerParams(
            dimension_semantics=("arbitrary", "arbitrary")),
    )(xf, topi, expert_w, expert_b.reshape(NE, 1, D))

    candidate_output = out.reshape(S, T, D)
    return candidate_output, beam_scores, expert_route
